# baseline (speedup 1.0000x reference)
import functools

import jax
import jax.numpy as jnp
from jax import lax
from jax.experimental import pallas as pl
from jax.experimental.pallas import tpu as pltpu
from jax.experimental.pallas import tpu_sc as plsc

V = 1000000
D = 64
L = 20
B = 16384
WD = 2 * D

_info = plsc.get_sparse_core_info()
NC, NS, LANES = _info.num_cores, _info.num_subcores, _info.num_lanes
NW = NC * NS
BPW = B // NW
HB = BPW // 2
NSUB = 2 * L
GCHUNK = 128
NGH = HB // GCHUNK
NGRP = BPW // LANES
QBLK = 8
NBLK = HB // (QBLK * LANES)


def _fire_gather(tblw, idxw, s, buf, sem):
    for k in range(NGH):
        pltpu.async_copy(
            tblw.at[idxw.at[pl.ds(s * HB + k * GCHUNK, GCHUNK)]],
            buf.at[pl.ds(k * GCHUNK, GCHUNK), :],
            sem,
        )


def _wait_gather(tblw, idxw, buf, sem):
    pltpu.make_async_copy(tblw.at[idxw.at[pl.ds(0, HB)]], buf, sem).wait()


def _compute_step(s, buf, idxm, acc_v, w1r_v, w2_v, lane_iota):
    l = s // 2
    half = s - 2 * l
    wl = w2_v[pl.ds(l, LANES)][0]

    def blk_body(bk, c):
        rbase0 = bk * (QBLK * LANES)
        row_idx = []
        offs = []
        idxs = []
        for q in range(QBLK):
            rb = rbase0 + q * LANES
            iv = idxm[pl.ds(s * HB + rb, LANES)]
            idxs.append(iv)
            offs.append((iv & 1) << 6)
            row_idx.append(rb + lane_iota)
        sacc = [jnp.zeros((LANES,), jnp.float32) for _ in range(QBLK)]
        colv = lane_iota
        for t in range(D):
            w = w1r_v[pl.ds(t * LANES, LANES)]
            for q in range(QBLK):
                v = plsc.load_gather(buf, [row_idx[q], colv + offs[q]])
                sacc[q] = sacc[q] + v * w
            colv = (colv + 1) & (D - 1)
        for q in range(QBLK):
            rb = rbase0 + q * LANES
            ab = half * HB + rb
            contrib = jnp.where(idxs[q] != 0, sacc[q], 0.0) * wl
            acc_v[pl.ds(ab, LANES)] = acc_v[pl.ds(ab, LANES)] + contrib
        return c

    lax.fori_loop(0, NBLK, blk_body, 0)


def _sc_body(idxf, tblw, w1r_h, w2h, out, idx_loc, idxm, idxw, buf0, buf1,
             acc_v, w1r_v, w2_v, sem0, sem1):
    wid = lax.axis_index("s") * NC + lax.axis_index("c")
    base = wid * BPW

    pltpu.sync_copy(idxf.at[pl.ds(base * L, BPW * L)], idx_loc)
    pltpu.sync_copy(w1r_h, w1r_v)
    pltpu.sync_copy(w2h, w2_v)

    lane_iota = lax.iota(jnp.int32, LANES)
    zeros = jnp.zeros((LANES,), jnp.float32)

    def tr_l(l, c):
        def tr_g(g, c2):
            flat = (g * LANES + lane_iota) * L + l
            v = plsc.load_gather(idx_loc, [flat])
            idxm[pl.ds(l * BPW + g * LANES, LANES)] = v
            idxw[pl.ds(l * BPW + g * LANES, LANES)] = lax.shift_right_logical(
                v, 1
            )
            return c2

        lax.fori_loop(0, NGRP, tr_g, 0)
        return c

    lax.fori_loop(0, L, tr_l, 0)

    def zero_body(g, c):
        acc_v[pl.ds(g * LANES, LANES)] = zeros
        return c

    lax.fori_loop(0, NGRP, zero_body, 0)

    _fire_gather(tblw, idxw, 0, buf0, sem0)
    _fire_gather(tblw, idxw, 1, buf1, sem1)

    def sub_pair(i, c):
        s0 = 2 * i

        _wait_gather(tblw, idxw, buf0, sem0)
        _compute_step(s0, buf0, idxm, acc_v, w1r_v, w2_v, lane_iota)

        @pl.when(i < (NSUB // 2 - 1))
        def _():
            _fire_gather(tblw, idxw, s0 + 2, buf0, sem0)

        _wait_gather(tblw, idxw, buf1, sem1)
        _compute_step(s0 + 1, buf1, idxm, acc_v, w1r_v, w2_v, lane_iota)

        @pl.when(i < (NSUB // 2 - 1))
        def _():
            _fire_gather(tblw, idxw, s0 + 3, buf1, sem1)

        return c

    lax.fori_loop(0, NSUB // 2, sub_pair, 0)

    const = w2_v[pl.ds(L, LANES)][0]

    def sig_body(g, c):
        x = acc_v[pl.ds(g * LANES, LANES)] + const
        acc_v[pl.ds(g * LANES, LANES)] = 1.0 / (1.0 + jnp.exp(-x))
        return c

    lax.fori_loop(0, NGRP, sig_body, 0)

    pltpu.sync_copy(acc_v, out.at[pl.ds(base, BPW)])


@functools.partial(
    pl.kernel,
    mesh=plsc.VectorSubcoreMesh(core_axis_name="c", subcore_axis_name="s"),
    out_type=jax.ShapeDtypeStruct((B,), jnp.float32),
    compiler_params=pltpu.CompilerParams(needs_layout_passes=False),
    scratch_types=[
        pltpu.VMEM((BPW * L,), jnp.int32),
        pltpu.VMEM((BPW * L,), jnp.int32),
        pltpu.VMEM((BPW * L,), jnp.int32),
        pltpu.VMEM((HB, WD), jnp.float32),
        pltpu.VMEM((HB, WD), jnp.float32),
        pltpu.VMEM((BPW,), jnp.float32),
        pltpu.VMEM((D * LANES,), jnp.float32),
        pltpu.VMEM((48,), jnp.float32),
        pltpu.SemaphoreType.DMA,
        pltpu.SemaphoreType.DMA,
    ],
)
def _sc_kernel(idxf, tblw, w1r_h, w2h, out, idx_loc, idxm, idxw, buf0, buf1,
               acc_v, w1r_v, w2_v, sem0, sem1):
    _sc_body(idxf, tblw, w1r_h, w2h, out, idx_loc, idxm, idxw, buf0, buf1,
             acc_v, w1r_v, w2_v, sem0, sem1)


def kernel(char_inputs, table, w1, b1, w2, b2):
    idxf = char_inputs.astype(jnp.int32).reshape(B * L)
    tblw = table.reshape(V // 2, WD)
    w1f = w1[:, 0]
    rot = (jnp.arange(D)[:, None] + jnp.arange(LANES)[None, :]) % D
    w1rot = w1f[rot].reshape(D * LANES)
    const = b1[0] * jnp.sum(w2) + b2[0]
    w2p = jnp.zeros((48,), jnp.float32).at[:L].set(w2[:, 0]).at[L].set(const)
    out = _sc_kernel(idxf, tblw, w1rot, w2p)
    return out.reshape(B, 1)


# speedup vs baseline: 2.8224x; 2.8224x over previous
import functools

import jax
import jax.numpy as jnp
from jax import lax
from jax.experimental import pallas as pl
from jax.experimental.pallas import tpu as pltpu
from jax.experimental.pallas import tpu_sc as plsc

V = 1000000
D = 64
L = 20
B = 16384

CBLK = 4096
NCOLB = (V + CBLK - 1) // CBLK
NROW = ((NCOLB + 7) // 8) * 8
VPAD = NROW * CBLK

_info = plsc.get_sparse_core_info()
NC, NS, LANES = _info.num_cores, _info.num_subcores, _info.num_lanes
NW = NC * NS
BPW = B // NW
GCHUNK = 128
NG = BPW // GCHUNK
NGRP = BPW // LANES


def _tc_body(tabT_ref, w1_ref, out_ref):
    j = lax.rem(pl.program_id(0), 8)
    prod = tabT_ref[...] * w1_ref[...]
    out_ref[pl.ds(j, 1), :] = jnp.sum(prod, axis=0, keepdims=True)


_tc_t1 = pl.pallas_call(
    _tc_body,
    grid=(NCOLB,),
    in_specs=[
        pl.BlockSpec((D, CBLK), lambda i: (0, i)),
        pl.BlockSpec((D, 1), lambda i: (0, 0)),
    ],
    out_specs=pl.BlockSpec((8, CBLK), lambda i: (i // 8, 0)),
    out_shape=jax.ShapeDtypeStruct((NROW, CBLK), jnp.float32),
)


def _fire_gather(t1, idxw, l, buf, sem):
    for k in range(NG):
        pltpu.async_copy(
            t1.at[idxw.at[pl.ds(l * BPW + k * GCHUNK, GCHUNK)]],
            buf.at[pl.ds(k * GCHUNK, GCHUNK), :],
            sem,
        )


def _wait_gather(t1, idxw, buf, sem):
    pltpu.make_async_copy(t1.at[idxw.at[pl.ds(0, BPW)]], buf, sem).wait()


def _compute_step(l, buf, idxm, acc_v, w2_v, lane_iota):
    wl = w2_v[pl.ds(l, LANES)][0]

    def g_body(g, c):
        rb = g * LANES
        idxs = idxm[pl.ds(l * BPW + rb, LANES)]
        colv = idxs & (LANES - 1)
        vals = plsc.load_gather(buf, [rb + lane_iota, colv])
        contrib = jnp.where(idxs != 0, vals, 0.0) * wl
        acc_v[pl.ds(rb, LANES)] = acc_v[pl.ds(rb, LANES)] + contrib
        return c

    lax.fori_loop(0, NGRP, g_body, 0)


def _sc_body(idxf, t1, w2h, out, idx_loc, idxm, idxw, buf0, buf1, acc_v,
             w2_v, sem0, sem1):
    wid = lax.axis_index("s") * NC + lax.axis_index("c")
    base = wid * BPW

    pltpu.sync_copy(idxf.at[pl.ds(base * L, BPW * L)], idx_loc)
    pltpu.sync_copy(w2h, w2_v)

    lane_iota = lax.iota(jnp.int32, LANES)
    zeros = jnp.zeros((LANES,), jnp.float32)

    def tr_l(l, c):
        def tr_g(g, c2):
            flat = (g * LANES + lane_iota) * L + l
            v = plsc.load_gather(idx_loc, [flat])
            idxm[pl.ds(l * BPW + g * LANES, LANES)] = v
            idxw[pl.ds(l * BPW + g * LANES, LANES)] = lax.shift_right_logical(
                v, 4
            )
            return c2

        lax.fori_loop(0, NGRP, tr_g, 0)
        return c

    lax.fori_loop(0, L, tr_l, 0)

    def zero_body(g, c):
        acc_v[pl.ds(g * LANES, LANES)] = zeros
        return c

    lax.fori_loop(0, NGRP, zero_body, 0)

    _fire_gather(t1, idxw, 0, buf0, sem0)
    _fire_gather(t1, idxw, 1, buf1, sem1)

    def l_pair(i, c):
        l0 = 2 * i

        _wait_gather(t1, idxw, buf0, sem0)
        _compute_step(l0, buf0, idxm, acc_v, w2_v, lane_iota)

        @pl.when(i < (L // 2 - 1))
        def _():
            _fire_gather(t1, idxw, l0 + 2, buf0, sem0)

        _wait_gather(t1, idxw, buf1, sem1)
        _compute_step(l0 + 1, buf1, idxm, acc_v, w2_v, lane_iota)

        @pl.when(i < (L // 2 - 1))
        def _():
            _fire_gather(t1, idxw, l0 + 3, buf1, sem1)

        return c

    lax.fori_loop(0, L // 2, l_pair, 0)

    const = w2_v[pl.ds(L, LANES)][0]

    def sig_body(g, c):
        x = acc_v[pl.ds(g * LANES, LANES)] + const
        acc_v[pl.ds(g * LANES, LANES)] = 1.0 / (1.0 + jnp.exp(-x))
        return c

    lax.fori_loop(0, NGRP, sig_body, 0)

    pltpu.sync_copy(acc_v, out.at[pl.ds(base, BPW)])


@functools.partial(
    pl.kernel,
    mesh=plsc.VectorSubcoreMesh(core_axis_name="c", subcore_axis_name="s"),
    out_type=jax.ShapeDtypeStruct((B,), jnp.float32),
    compiler_params=pltpu.CompilerParams(
        needs_layout_passes=False, use_tc_tiling_on_sc=False
    ),
    scratch_types=[
        pltpu.VMEM((BPW * L,), jnp.int32),
        pltpu.VMEM((BPW * L,), jnp.int32),
        pltpu.VMEM((BPW * L,), jnp.int32),
        pltpu.VMEM((BPW, LANES), jnp.float32),
        pltpu.VMEM((BPW, LANES), jnp.float32),
        pltpu.VMEM((BPW,), jnp.float32),
        pltpu.VMEM((48,), jnp.float32),
        pltpu.SemaphoreType.DMA,
        pltpu.SemaphoreType.DMA,
    ],
)
def _sc_kernel(idxf, t1, w2h, out, idx_loc, idxm, idxw, buf0, buf1, acc_v,
               w2_v, sem0, sem1):
    _sc_body(idxf, t1, w2h, out, idx_loc, idxm, idxw, buf0, buf1, acc_v,
             w2_v, sem0, sem1)


def kernel(char_inputs, table, w1, b1, w2, b2):
    idxf = char_inputs.astype(jnp.int32).reshape(B * L)
    tabT = jnp.transpose(table)
    t1 = _tc_t1(tabT, w1).reshape(VPAD // LANES, LANES)
    const = b1[0] * jnp.sum(w2) + b2[0]
    w2p = jnp.zeros((48,), jnp.float32).at[:L].set(w2[:, 0]).at[L].set(const)
    out = _sc_kernel(idxf, t1, w2p)
    return out.reshape(B, 1)


# speedup vs baseline: 3.8507x; 1.3643x over previous
import functools

import jax
import jax.numpy as jnp
from jax import lax
from jax.experimental import pallas as pl
from jax.experimental.pallas import tpu as pltpu
from jax.experimental.pallas import tpu_sc as plsc

V = 1000000
D = 64
L = 20
B = 16384

CBLK = 8192
NCOLB = (V + CBLK - 1) // CBLK
NROW = ((NCOLB + 7) // 8) * 8
VPAD = NROW * CBLK

_info = plsc.get_sparse_core_info()
NC, NS, LANES = _info.num_cores, _info.num_subcores, _info.num_lanes
NW = NC * NS
BPW = B // NW
GCHUNK = 128
NG = BPW // GCHUNK
NGRP = BPW // LANES


def _tc_body(tabT_ref, w1T_ref, out_ref):
    j = lax.rem(pl.program_id(0), 8)
    out_ref[pl.ds(j, 1), :] = jnp.dot(
        w1T_ref[...], tabT_ref[...], preferred_element_type=jnp.float32
    )


_tc_t1 = pl.pallas_call(
    _tc_body,
    grid=(NCOLB,),
    in_specs=[
        pl.BlockSpec((D, CBLK), lambda i: (0, i)),
        pl.BlockSpec((1, D), lambda i: (0, 0)),
    ],
    out_specs=pl.BlockSpec((8, CBLK), lambda i: (i // 8, 0)),
    out_shape=jax.ShapeDtypeStruct((NROW, CBLK), jnp.float32),
)


def _fire_gather(t1, idxw, l, buf, sem):
    for k in range(NG):
        pltpu.async_copy(
            t1.at[idxw.at[pl.ds(l * BPW + k * GCHUNK, GCHUNK)]],
            buf.at[pl.ds(k * GCHUNK, GCHUNK), :],
            sem,
        )


def _wait_gather(t1, idxw, buf, sem):
    pltpu.make_async_copy(t1.at[idxw.at[pl.ds(0, BPW)]], buf, sem).wait()


def _compute_step(l, buf, idxm, acc_v, w2_v, lane_iota):
    wl = w2_v[pl.ds(l, LANES)][0]

    def g_body(g, c):
        rb = g * LANES
        idxs = idxm[pl.ds(l * BPW + rb, LANES)]
        colv = idxs & (LANES - 1)
        vals = plsc.load_gather(buf, [rb + lane_iota, colv])
        contrib = jnp.where(idxs != 0, vals, 0.0) * wl
        acc_v[pl.ds(rb, LANES)] = acc_v[pl.ds(rb, LANES)] + contrib
        return c

    lax.fori_loop(0, NGRP, g_body, 0)


def _sc_body(idxf, t1, w2h, out, idx_loc, idxm, idxw, buf0, buf1, acc_v,
             w2_v, sem0, sem1):
    wid = lax.axis_index("s") * NC + lax.axis_index("c")
    base = wid * BPW

    pltpu.sync_copy(idxf.at[pl.ds(base * L, BPW * L)], idx_loc)
    pltpu.sync_copy(w2h, w2_v)

    lane_iota = lax.iota(jnp.int32, LANES)
    zeros = jnp.zeros((LANES,), jnp.float32)

    def tr_l(l, c):
        def tr_g(g, c2):
            flat = (g * LANES + lane_iota) * L + l
            v = plsc.load_gather(idx_loc, [flat])
            idxm[pl.ds(l * BPW + g * LANES, LANES)] = v
            idxw[pl.ds(l * BPW + g * LANES, LANES)] = lax.shift_right_logical(
                v, 4
            )
            return c2

        lax.fori_loop(0, NGRP, tr_g, 0)
        return c

    lax.fori_loop(0, L, tr_l, 0)

    def zero_body(g, c):
        acc_v[pl.ds(g * LANES, LANES)] = zeros
        return c

    lax.fori_loop(0, NGRP, zero_body, 0)

    _fire_gather(t1, idxw, 0, buf0, sem0)
    _fire_gather(t1, idxw, 1, buf1, sem1)

    def l_pair(i, c):
        l0 = 2 * i

        _wait_gather(t1, idxw, buf0, sem0)
        _compute_step(l0, buf0, idxm, acc_v, w2_v, lane_iota)

        @pl.when(i < (L // 2 - 1))
        def _():
            _fire_gather(t1, idxw, l0 + 2, buf0, sem0)

        _wait_gather(t1, idxw, buf1, sem1)
        _compute_step(l0 + 1, buf1, idxm, acc_v, w2_v, lane_iota)

        @pl.when(i < (L // 2 - 1))
        def _():
            _fire_gather(t1, idxw, l0 + 3, buf1, sem1)

        return c

    lax.fori_loop(0, L // 2, l_pair, 0)

    const = w2_v[pl.ds(L, LANES)][0]

    def sig_body(g, c):
        x = acc_v[pl.ds(g * LANES, LANES)] + const
        acc_v[pl.ds(g * LANES, LANES)] = 1.0 / (1.0 + jnp.exp(-x))
        return c

    lax.fori_loop(0, NGRP, sig_body, 0)

    pltpu.sync_copy(acc_v, out.at[pl.ds(base, BPW)])


@functools.partial(
    pl.kernel,
    mesh=plsc.VectorSubcoreMesh(core_axis_name="c", subcore_axis_name="s"),
    out_type=jax.ShapeDtypeStruct((B,), jnp.float32),
    compiler_params=pltpu.CompilerParams(
        needs_layout_passes=False, use_tc_tiling_on_sc=False
    ),
    scratch_types=[
        pltpu.VMEM((BPW * L,), jnp.int32),
        pltpu.VMEM((BPW * L,), jnp.int32),
        pltpu.VMEM((BPW * L,), jnp.int32),
        pltpu.VMEM((BPW, LANES), jnp.float32),
        pltpu.VMEM((BPW, LANES), jnp.float32),
        pltpu.VMEM((BPW,), jnp.float32),
        pltpu.VMEM((48,), jnp.float32),
        pltpu.SemaphoreType.DMA,
        pltpu.SemaphoreType.DMA,
    ],
)
def _sc_kernel(idxf, t1, w2h, out, idx_loc, idxm, idxw, buf0, buf1, acc_v,
               w2_v, sem0, sem1):
    _sc_body(idxf, t1, w2h, out, idx_loc, idxm, idxw, buf0, buf1, acc_v,
             w2_v, sem0, sem1)


def kernel(char_inputs, table, w1, b1, w2, b2):
    idxf = char_inputs.astype(jnp.int32).reshape(B * L)
    tabT = jnp.transpose(table)
    w1T = jnp.transpose(w1)
    t1 = _tc_t1(tabT, w1T).reshape(VPAD // LANES, LANES)
    const = b1[0] * jnp.sum(w2) + b2[0]
    w2p = jnp.zeros((48,), jnp.float32).at[:L].set(w2[:, 0]).at[L].set(const)
    out = _sc_kernel(idxf, t1, w2p)
    return out.reshape(B, 1)
